# baseline (speedup 1.0000x reference)
import functools

import jax
import jax.numpy as jnp
from jax import lax
from jax.experimental import pallas as pl
from jax.experimental.pallas import tpu as pltpu
from jax.experimental.pallas import tpu_sc as plsc

NC = 2
NS = 16
L = 16
NW = NC * NS


@functools.lru_cache(maxsize=None)
def _build(rows, n_chunks, chunk_size, batch):
    assert n_chunks * chunk_size == 64 and chunk_size == L
    bpw = batch // NW
    n_idx = bpw * n_chunks
    n_grp = n_idx // L
    n_dma = n_idx // 128
    ids_per_grp = L // n_chunks

    mesh = plsc.VectorSubcoreMesh(core_axis_name="c", subcore_axis_name="s")

    @functools.partial(
        pl.kernel,
        out_type=jax.ShapeDtypeStruct((batch * n_chunks, chunk_size),
                                      jnp.float32),
        mesh=mesh,
        scratch_types=[
            pltpu.VMEM((bpw,), jnp.int32),
            pltpu.VMEM((L,), jnp.int32),
            pltpu.VMEM((n_dma, 128), jnp.int32),
            pltpu.VMEM((n_idx, chunk_size), jnp.float32),
            pltpu.SemaphoreType.DMA,
        ],
        compiler_params=pltpu.CompilerParams(needs_layout_passes=False,
                                             use_tc_tiling_on_sc=False),
    )
    def sc_kernel(table_h, x_h, pat_h, out_h, x_v, pat_v, idx_v, rows_v, sem):
        wid = lax.axis_index("s") * NC + lax.axis_index("c")
        base = wid * bpw
        pltpu.sync_copy(x_h.at[pl.ds(base, bpw)], x_v)
        pltpu.sync_copy(pat_h, pat_v)

        lanes = lax.iota(jnp.int32, L)
        sub = lax.shift_right_logical(lanes, 2)
        cofs = plsc.bitcast(lanes & 3, jnp.uint32)
        coeff = plsc.bitcast(pat_v[...], jnp.uint32)

        def hash_body(g, carry):
            xg = plsc.load_gather(x_v, [g * ids_per_grp + sub])
            xu = plsc.bitcast(xg, jnp.uint32)
            h = (xu * coeff) % jnp.uint32(rows)
            j = h * jnp.uint32(n_chunks) + cofs
            idx_v[g // 8, pl.ds((g % 8) * L, L)] = plsc.bitcast(j, jnp.int32)
            return carry

        lax.fori_loop(0, n_grp, hash_body, 0)

        copies = [
            pltpu.async_copy(table_h.at[idx_v.at[d]],
                             rows_v.at[pl.ds(d * 128, 128)], sem)
            for d in range(n_dma)
        ]
        for c in copies:
            c.wait()
        pltpu.sync_copy(rows_v, out_h.at[pl.ds(base * n_chunks, n_idx)])

    return sc_kernel


def kernel(x, table, hash_coeffs):
    rows, n_chunks, chunk_size = table.shape
    batch = x.shape[0]
    table_flat = table.reshape(rows * n_chunks, chunk_size)
    pat = lax.bitcast_convert_type(
        jnp.tile(hash_coeffs, L // n_chunks), jnp.int32)
    out = _build(rows, n_chunks, chunk_size, batch)(
        table_flat, x.astype(jnp.int32), pat)
    return out.reshape(batch, n_chunks * chunk_size)


# speedup vs baseline: 2.3489x; 2.3489x over previous
import functools

import jax
import jax.numpy as jnp
from jax import lax
from jax.experimental import pallas as pl
from jax.experimental.pallas import tpu as pltpu
from jax.experimental.pallas import tpu_sc as plsc

NC = 2
NS = 16
L = 16
NW = NC * NS


@functools.lru_cache(maxsize=None)
def _build(rows, n_chunks, chunk_size, batch):
    assert n_chunks == 4 and chunk_size == L
    dim = n_chunks * chunk_size
    bpw = batch // NW
    n_vec = bpw // L
    n_bt = bpw // 128

    mesh = plsc.VectorSubcoreMesh(core_axis_name="c", subcore_axis_name="s")

    @functools.partial(
        pl.kernel,
        out_type=jax.ShapeDtypeStruct((8, batch // 128, 8, 128),
                                      jnp.float32),
        mesh=mesh,
        scratch_types=[
            pltpu.VMEM((bpw,), jnp.int32),
            pltpu.VMEM((n_chunks, L), jnp.int32),
            pltpu.VMEM((n_chunks, bpw), jnp.int32),
            pltpu.VMEM((2, 128, dim), jnp.float32),
            pltpu.VMEM((8, n_bt, 8, 128), jnp.float32),
            pltpu.SemaphoreType.DMA,
            pltpu.SemaphoreType.DMA,
            pltpu.SemaphoreType.DMA,
        ],
        compiler_params=pltpu.CompilerParams(needs_layout_passes=False,
                                             use_tc_tiling_on_sc=False),
    )
    def sc_kernel(tbl_h, x_h, pat_h, out_h, x_v, pat_v, r_v, gbuf_v, slab_v,
                  sem_a, sem_b, gsem):
        wid = lax.axis_index("s") * NC + lax.axis_index("c")
        base = wid * bpw
        pltpu.sync_copy(x_h.at[pl.ds(base, bpw)], x_v)
        pltpu.sync_copy(pat_h, pat_v)

        def hash_body(g, carry):
            xu = plsc.bitcast(x_v[pl.ds(g * L, L)], jnp.uint32)
            for c in range(n_chunks):
                cvec = plsc.bitcast(pat_v[c, :], jnp.uint32)
                h = (xu * cvec) % jnp.uint32(rows)
                r_v[c, pl.ds(g * L, L)] = plsc.bitcast(h, jnp.int32)
            return carry

        lax.fori_loop(0, n_vec, hash_body, 0)

        n_dma = n_chunks * n_bt

        def fire(d, buf, sem):
            c = d // n_bt
            bt = d % n_bt
            pltpu.async_copy(
                tbl_h.at[r_v.at[c, pl.ds(bt * 128, 128)]],
                gbuf_v.at[buf], sem)

        def wait(sem):
            pltpu.make_async_copy(
                tbl_h.at[pl.ds(0, 128)], gbuf_v.at[0], sem).wait()

        lanes = lax.iota(jnp.int32, L)

        def extract(d, buf):
            c = d // n_bt
            bt = d % n_bt
            for k in range(L):
                jt = 2 * c + k // 8
                js = k % 8
                kvec = lanes * 0 + (c * L + k)
                for v in range(128 // L):
                    evec = jnp.int32(v * L) + lanes
                    vals = plsc.load_gather(gbuf_v.at[buf], [evec, kvec])
                    slab_v[jt, bt, js, pl.ds(v * L, L)] = vals

        fire(0, 0, sem_a)
        fire(1, 1, sem_b)

        def pipe_body(m, carry):
            d0 = 2 * m
            wait(sem_a)
            extract(d0, 0)

            @pl.when(d0 + 2 < n_dma)
            def _next_a():
                fire(d0 + 2, 0, sem_a)

            wait(sem_b)
            extract(d0 + 1, 1)

            @pl.when(d0 + 3 < n_dma)
            def _next_b():
                fire(d0 + 3, 1, sem_b)

            return carry

        lax.fori_loop(0, n_dma // 2, pipe_body, 0)

        pltpu.async_copy(
            slab_v, out_h.at[:, pl.ds(n_bt * wid, n_bt), :, :], gsem).wait()

    return sc_kernel


def kernel(x, table, hash_coeffs):
    rows, n_chunks, chunk_size = table.shape
    batch = x.shape[0]
    tbl2 = table.reshape(rows, n_chunks * chunk_size)
    pat = lax.bitcast_convert_type(
        jnp.broadcast_to(hash_coeffs[:, None], (n_chunks, L)), jnp.int32)
    out4 = _build(rows, n_chunks, chunk_size, batch)(
        tbl2, x.astype(jnp.int32), pat)
    out2 = out4.transpose(0, 2, 1, 3).reshape(n_chunks * chunk_size, batch)
    return out2.T
